# baseline (speedup 1.0000x reference)
import functools

import jax
import jax.numpy as jnp
import numpy as np
from jax import lax
from jax.experimental import pallas as pl
from jax.experimental.pallas import tpu as pltpu
from jax.experimental.pallas import tpu_sc as plsc

F = 26
V = 100000
D = 16
B = 16384
TOTAL = B * F
NC, NS, L = 2, 16, 16
NW = NC * NS
PER_W = TOTAL // NW
CHUNK = 1664
ROWS = CHUNK // F
NCH = PER_W // CHUNK
GSZ = 128
NG = CHUNK // GSZ

_mesh = plsc.VectorSubcoreMesh(core_axis_name="c", subcore_axis_name="s")


@functools.partial(
    pl.kernel,
    mesh=_mesh,
    compiler_params=pltpu.CompilerParams(
        use_tc_tiling_on_sc=False, needs_layout_passes=False
    ),
    out_type=jax.ShapeDtypeStruct((TOTAL, D), jnp.float32),
    scratch_types=[
        pltpu.VMEM((ROWS, F), jnp.int32),
        pltpu.VMEM((CHUNK,), jnp.int32),
        pltpu.VMEM((CHUNK,), jnp.int32),
        pltpu.VMEM((CHUNK,), jnp.int32),
        pltpu.VMEM((CHUNK,), jnp.int32),
        pltpu.VMEM((CHUNK, D), jnp.float32),
        pltpu.SemaphoreType.DMA,
    ],
)
def _gather_kernel(x_hbm, rt_hbm, ct_hbm, ot_hbm, table_hbm, out_hbm,
                   x_v, rt_v, ct_v, ot_v, idx_v, rows_v, sem):
    wid = lax.axis_index("s") * NC + lax.axis_index("c")
    base = wid * PER_W

    pltpu.sync_copy(rt_hbm, rt_v)
    pltpu.sync_copy(ct_hbm, ct_v)
    pltpu.sync_copy(ot_hbm, ot_v)

    for c in range(NCH):
        start = base + c * CHUNK
        r0 = start // F
        pltpu.sync_copy(x_hbm.at[pl.ds(r0, ROWS), :], x_v)

        def vec_body(k, carry):
            s = pl.ds(k * L, L)
            vals = plsc.load_gather(x_v, [rt_v[s], ct_v[s]])
            idx_v[s] = vals + ot_v[s]
            return carry

        lax.fori_loop(0, CHUNK // L, vec_body, 0)

        gathers = []
        for j in range(NG):
            s = pl.ds(j * GSZ, GSZ)
            gathers.append(
                pltpu.async_copy(table_hbm.at[idx_v.at[s]], rows_v.at[s], sem)
            )
        for d in gathers:
            d.wait()

        pltpu.sync_copy(rows_v, out_hbm.at[pl.ds(start, CHUNK)])


_POS = np.arange(CHUNK, dtype=np.int32)
_RTAB = jnp.asarray(_POS // F)
_CTAB = jnp.asarray(_POS % F)
_OTAB = jnp.asarray((_POS % F) * V)


def kernel(x, tables):
    flat_tables = tables.reshape(F * V, D)
    out = _gather_kernel(x.astype(jnp.int32), _RTAB, _CTAB, _OTAB, flat_tables)
    return out.reshape(B, F * D)


# speedup vs baseline: 1.0031x; 1.0031x over previous
import functools

import jax
import jax.numpy as jnp
import numpy as np
from jax import lax
from jax.experimental import pallas as pl
from jax.experimental.pallas import tpu as pltpu
from jax.experimental.pallas import tpu_sc as plsc

F = 26
V = 100000
D = 16
B = 16384
TOTAL = B * F
NC, NS, L = 2, 16, 16
NW = NC * NS
PER_W = TOTAL // NW
CHUNK = 1664
ROWS = CHUNK // F
NCH = PER_W // CHUNK
GSZ = 128
NG = CHUNK // GSZ

_mesh = plsc.VectorSubcoreMesh(core_axis_name="c", subcore_axis_name="s")


@functools.partial(
    pl.kernel,
    mesh=_mesh,
    compiler_params=pltpu.CompilerParams(
        use_tc_tiling_on_sc=False, needs_layout_passes=False
    ),
    out_type=jax.ShapeDtypeStruct((TOTAL, D), jnp.float32),
    scratch_types=[
        pltpu.VMEM((ROWS, F), jnp.int32),
        pltpu.VMEM((CHUNK,), jnp.int32),
        pltpu.VMEM((CHUNK,), jnp.int32),
        pltpu.VMEM((CHUNK,), jnp.int32),
        pltpu.VMEM((CHUNK,), jnp.int32),
        pltpu.VMEM((CHUNK, D), jnp.float32),
        pltpu.SemaphoreType.DMA,
    ],
)
def _gather_kernel(x_hbm, rt_hbm, ct_hbm, ot_hbm, table_hbm, out_hbm,
                   x_v, rt_v, ct_v, ot_v, idx_v, rows_v, sem):
    wid = lax.axis_index("s") * NC + lax.axis_index("c")
    base = wid * PER_W

    pltpu.sync_copy(rt_hbm, rt_v)
    pltpu.sync_copy(ct_hbm, ct_v)
    pltpu.sync_copy(ot_hbm, ot_v)

    for c in range(NCH):
        start = base + c * CHUNK
        r0 = start // F
        pltpu.sync_copy(x_hbm.at[pl.ds(r0, ROWS), :], x_v)

        def vec_body(k, carry):
            s = pl.ds(k * L, L)
            vals = plsc.load_gather(x_v, [rt_v[s], ct_v[s]])
            idx_v[s] = vals + ot_v[s]
            return carry

        lax.fori_loop(0, CHUNK // L, vec_body, 0)

        gathers = []
        for j in range(NG):
            s = pl.ds(j * GSZ, GSZ)
            gathers.append(
                pltpu.async_copy(table_hbm.at[idx_v.at[s]], rows_v.at[s], sem)
            )
        for d in gathers:
            d.wait()

        pltpu.sync_copy(rows_v, out_hbm.at[pl.ds(start, CHUNK)])


_POS = np.arange(CHUNK, dtype=np.int32)
_RTAB = _POS // F
_CTAB = _POS % F
_OTAB = (_POS % F) * V


def kernel(x, tables):
    flat_tables = tables.reshape(F * V, D)
    out = _gather_kernel(
        x.astype(jnp.int32),
        jnp.asarray(_RTAB),
        jnp.asarray(_CTAB),
        jnp.asarray(_OTAB),
        flat_tables,
    )
    return out.reshape(B, F * D)
